# baseline (speedup 1.0000x reference)
import functools

import jax
import jax.numpy as jnp
from jax import lax
from jax.experimental import pallas as pl
from jax.experimental.pallas import tpu as pltpu
from jax.experimental.pallas import tpu_sc as plsc

VOCAB = 100000
EMBED = 32
BATCH = 1024
CTX = 20

NC = 2
NS = 16
NW = NC * NS
RPW = BATCH // NW
IPW = RPW * CTX
CHUNK = 128
NCHUNK = IPW // CHUNK

TV = 2048
NV = (VOCAB + TV - 1) // TV


def _sc_avg_body(table_hbm, idx_hbm, out_hbm, idx_v, rows_v, avg_v, sem):
    wid = lax.axis_index("s") * NC + lax.axis_index("c")
    pltpu.sync_copy(idx_hbm.at[wid], idx_v)
    copies = [
        pltpu.async_copy(
            table_hbm.at[idx_v.at[k]],
            rows_v.at[pl.ds(k * CHUNK, CHUNK)],
            sem,
        )
        for k in range(NCHUNK)
    ]
    for c in copies:
        c.wait()

    def row_body(r, carry):
        base = r * CTX
        for h in range(EMBED // 16):
            acc = rows_v[base, pl.ds(h * 16, 16)]
            for c in range(1, CTX):
                acc = acc + rows_v[base + c, pl.ds(h * 16, 16)]
            avg_v[r, pl.ds(h * 16, 16)] = acc * (1.0 / CTX)
        return carry

    lax.fori_loop(0, RPW, row_body, 0)
    pltpu.sync_copy(avg_v, out_hbm.at[pl.ds(wid * RPW, RPW)])


@functools.cache
def _sc_avg():
    return pl.kernel(
        _sc_avg_body,
        mesh=plsc.VectorSubcoreMesh(core_axis_name="c", subcore_axis_name="s"),
        out_type=jax.ShapeDtypeStruct((BATCH, EMBED), jnp.float32),
        scratch_types=[
            pltpu.VMEM((NCHUNK, CHUNK), jnp.int32),
            pltpu.VMEM((IPW, EMBED), jnp.float32),
            pltpu.VMEM((RPW, EMBED), jnp.float32),
            pltpu.SemaphoreType.DMA,
        ],
        compiler_params=pltpu.CompilerParams(use_tc_tiling_on_sc=False),
    )


def _tc_softmax_body(avg_ref, w_ref, b_ref, o_ref, m_ref, s_ref):
    p = pl.program_id(0)
    j = pl.program_id(1)
    logits = (
        jnp.dot(avg_ref[...], w_ref[...], preferred_element_type=jnp.float32)
        + b_ref[...]
    )
    cols = j * TV + lax.broadcasted_iota(jnp.int32, logits.shape, 1)
    logits = jnp.where(cols < VOCAB, logits, -jnp.inf)

    @pl.when(p == 0)
    def _phase0():
        @pl.when(j == 0)
        def _init():
            m_ref[...] = jnp.full(m_ref.shape, -jnp.inf, m_ref.dtype)
            s_ref[...] = jnp.zeros(s_ref.shape, s_ref.dtype)

        m_old = m_ref[...]
        m_new = jnp.maximum(m_old, jnp.max(logits, axis=1, keepdims=True))
        s_ref[...] = s_ref[...] * jnp.exp(m_old - m_new) + jnp.sum(
            jnp.exp(logits - m_new), axis=1, keepdims=True
        )
        m_ref[...] = m_new

    @pl.when(p == 1)
    def _phase1():
        o_ref[...] = jnp.exp(logits - m_ref[...]) / s_ref[...]


def _tc_softmax(avg, W, b2):
    return pl.pallas_call(
        _tc_softmax_body,
        grid=(2, NV),
        in_specs=[
            pl.BlockSpec((BATCH, EMBED), lambda p, j: (0, 0)),
            pl.BlockSpec((EMBED, TV), lambda p, j: (0, j)),
            pl.BlockSpec((1, TV), lambda p, j: (0, j)),
        ],
        out_specs=pl.BlockSpec((BATCH, TV), lambda p, j: (0, p * j)),
        out_shape=jax.ShapeDtypeStruct((BATCH, VOCAB), jnp.float32),
        scratch_shapes=[
            pltpu.VMEM((BATCH, 1), jnp.float32),
            pltpu.VMEM((BATCH, 1), jnp.float32),
        ],
        compiler_params=pltpu.CompilerParams(
            dimension_semantics=("arbitrary", "arbitrary"),
        ),
    )(avg, W, b2)


def kernel(inputs, E, W, b):
    idx = inputs.astype(jnp.int32).reshape(NW, NCHUNK, CHUNK)
    avg = _sc_avg()(E, idx)
    return _tc_softmax(avg, W, b.reshape(1, VOCAB))
